# baseline (speedup 1.0000x reference)
import functools

import jax
import jax.numpy as jnp
from jax import lax
from jax.experimental import pallas as pl
from jax.experimental.pallas import tpu as pltpu
from jax.experimental.pallas import tpu_sc as plsc


_CH = 128


@functools.lru_cache(maxsize=None)
def _build(n_workers: int, n_cores: int, b_tot: int, vocab: int, d: int):
    b_per_w = b_tot // n_workers
    n_ch = b_per_w // _CH
    mesh = plsc.VectorSubcoreMesh(core_axis_name="c", subcore_axis_name="s")

    @functools.partial(
        pl.kernel,
        mesh=mesh,
        out_type=jax.ShapeDtypeStruct((b_tot, d), jnp.float32),
        compiler_params=pltpu.CompilerParams(use_tc_tiling_on_sc=False),
        scratch_types=[
            pltpu.VMEM((n_ch, _CH), jnp.int32),
            pltpu.VMEM((_CH, d), jnp.float32),
            pltpu.SemaphoreType.DMA,
        ],
    )
    def emb(x_hbm, table_hbm, out_hbm, idx_v, rows_v, sem):
        wid = lax.axis_index("s") * n_cores + lax.axis_index("c")
        base = wid * b_per_w
        pltpu.sync_copy(x_hbm.at[wid], idx_v)

        def body(c, carry):
            pltpu.async_copy(table_hbm.at[idx_v.at[c]], rows_v, sem).wait()
            pltpu.sync_copy(rows_v, out_hbm.at[pl.ds(base + c * _CH, _CH)])
            return carry

        lax.fori_loop(0, n_ch, body, 0)

    return emb


def kernel(x, table):
    b, s = x.shape
    vocab, d = table.shape
    info = plsc.get_sparse_core_info()
    n_workers = info.num_cores * info.num_subcores
    b_tot = b * s
    b_per_w = b_tot // n_workers
    xf = x.reshape(n_workers, b_per_w // _CH, _CH)
    emb = _build(n_workers, info.num_cores, b_tot, vocab, d)
    out = emb(xf, table)
    return out.reshape(b, s, d)


# speedup vs baseline: 1.1137x; 1.1137x over previous
import functools

import jax
import jax.numpy as jnp
from jax import lax
from jax.experimental import pallas as pl
from jax.experimental.pallas import tpu as pltpu
from jax.experimental.pallas import tpu_sc as plsc


_CH = 128
_K = 4


@functools.lru_cache(maxsize=None)
def _build(n_workers: int, n_cores: int, b_tot: int, vocab: int, d: int):
    b_per_w = b_tot // n_workers
    n_ch = b_per_w // _CH
    n_pair = n_ch // (2 * _K)
    grp = _K * _CH
    mesh = plsc.VectorSubcoreMesh(core_axis_name="c", subcore_axis_name="s")

    @functools.partial(
        pl.kernel,
        mesh=mesh,
        out_type=jax.ShapeDtypeStruct((b_tot, d), jnp.float32),
        compiler_params=pltpu.CompilerParams(use_tc_tiling_on_sc=False),
        scratch_types=[
            pltpu.VMEM((n_ch, _CH), jnp.int32),
            pltpu.VMEM((grp, d), jnp.float32),
            pltpu.VMEM((grp, d), jnp.float32),
            pltpu.SemaphoreType.DMA,
            pltpu.SemaphoreType.DMA,
            pltpu.SemaphoreType.DMA,
            pltpu.SemaphoreType.DMA,
        ],
    )
    def emb(x_hbm, table_hbm, out_hbm, idx_v, rows_a, rows_b,
            sem_ga, sem_gb, sem_oa, sem_ob):
        wid = lax.axis_index("s") * n_cores + lax.axis_index("c")
        base = wid * b_per_w
        pltpu.sync_copy(x_hbm.at[wid], idx_v)

        def gather_group(c0, rows_v, sem):
            return [
                pltpu.make_async_copy(
                    table_hbm.at[idx_v.at[c0 + j]],
                    rows_v.at[pl.ds(j * _CH, _CH)], sem)
                for j in range(_K)
            ]

        def out_group(c0, rows_v, sem):
            return pltpu.make_async_copy(
                rows_v, out_hbm.at[pl.ds(base + c0 * _CH, grp)], sem)

        for cp in gather_group(0, rows_a, sem_ga):
            cp.start()

        def body(u, carry):
            c0 = u * 2 * _K

            @pl.when(u > 0)
            def _():
                out_group(c0 - _K, rows_b, sem_ob).wait()

            for cp in gather_group(c0 + _K, rows_b, sem_gb):
                cp.start()

            for cp in gather_group(c0, rows_a, sem_ga):
                cp.wait()
            out_group(c0, rows_a, sem_oa).start()
            out_group(c0, rows_a, sem_oa).wait()

            @pl.when(u < n_pair - 1)
            def _():
                for cp in gather_group(c0 + 2 * _K, rows_a, sem_ga):
                    cp.start()

            for cp in gather_group(c0 + _K, rows_b, sem_gb):
                cp.wait()
            out_group(c0 + _K, rows_b, sem_ob).start()
            return carry

        lax.fori_loop(0, n_pair, body, 0)
        out_group(n_ch - _K, rows_b, sem_ob).wait()

    return emb


def kernel(x, table):
    b, s = x.shape
    vocab, d = table.shape
    info = plsc.get_sparse_core_info()
    n_workers = info.num_cores * info.num_subcores
    b_tot = b * s
    b_per_w = b_tot // n_workers
    xf = x.reshape(n_workers, b_per_w // _CH, _CH)
    emb = _build(n_workers, info.num_cores, b_tot, vocab, d)
    out = emb(xf, table)
    return out.reshape(b, s, d)


# speedup vs baseline: 1.1143x; 1.0006x over previous
import functools

import jax
import jax.numpy as jnp
from jax import lax
from jax.experimental import pallas as pl
from jax.experimental.pallas import tpu as pltpu
from jax.experimental.pallas import tpu_sc as plsc


_CH = 128
_K = 2
_NBUF = 5
_LAG = 3


@functools.lru_cache(maxsize=None)
def _build(n_workers: int, n_cores: int, b_tot: int, vocab: int, d: int):
    b_per_w = b_tot // n_workers
    n_ch = b_per_w // _CH
    n_grp = n_ch // _K
    rounds = n_grp // _NBUF
    grp = _K * _CH
    mesh = plsc.VectorSubcoreMesh(core_axis_name="c", subcore_axis_name="s")

    scratch = [pltpu.VMEM((n_ch, _CH), jnp.int32)]
    scratch += [pltpu.VMEM((grp, d), jnp.float32) for _ in range(_NBUF)]
    scratch += [pltpu.SemaphoreType.DMA for _ in range(2 * _NBUF)]

    @functools.partial(
        pl.kernel,
        mesh=mesh,
        out_type=jax.ShapeDtypeStruct((b_tot, d), jnp.float32),
        compiler_params=pltpu.CompilerParams(use_tc_tiling_on_sc=False),
        scratch_types=scratch,
    )
    def emb(x_hbm, table_hbm, out_hbm, idx_v, *rest):
        bufs = rest[:_NBUF]
        gsem = rest[_NBUF:2 * _NBUF]
        wsem = rest[2 * _NBUF:]
        wid = lax.axis_index("s") * n_cores + lax.axis_index("c")
        base = wid * b_per_w
        pltpu.sync_copy(x_hbm.at[wid], idx_v)

        def g_copies(g, b):
            return [
                pltpu.make_async_copy(
                    table_hbm.at[idx_v.at[g * _K + j]],
                    bufs[b].at[pl.ds(j * _CH, _CH)], gsem[b])
                for j in range(_K)
            ]

        def wb_copy(g, b):
            return pltpu.make_async_copy(
                bufs[b], out_hbm.at[pl.ds(base + g * grp, grp)], wsem[b])

        for b in range(_LAG):
            for cp in g_copies(b, b):
                cp.start()

        def body(r, carry):
            g0 = r * _NBUF
            for i in range(_NBUF):
                g = g0 + i
                fg = g + _LAG
                fb = (i + _LAG) % _NBUF

                @pl.when(jnp.logical_and(fg >= _NBUF, fg < n_grp))
                def _():
                    wb_copy(fg - _NBUF, fb).wait()

                @pl.when(fg < n_grp)
                def _():
                    for cp in g_copies(fg, fb):
                        cp.start()

                for cp in g_copies(g, i):
                    cp.wait()
                wb_copy(g, i).start()
            return carry

        lax.fori_loop(0, rounds, body, 0)
        for i in range(_NBUF):
            wb_copy(n_grp - _NBUF + i, i).wait()

    return emb


def kernel(x, table):
    b, s = x.shape
    vocab, d = table.shape
    info = plsc.get_sparse_core_info()
    n_workers = info.num_cores * info.num_subcores
    b_tot = b * s
    b_per_w = b_tot // n_workers
    xf = x.reshape(n_workers, b_per_w // _CH, _CH)
    emb = _build(n_workers, info.num_cores, b_tot, vocab, d)
    out = emb(xf, table)
    return out.reshape(b, s, d)


# speedup vs baseline: 1.1155x; 1.0010x over previous
import functools

import jax
import jax.numpy as jnp
from jax import lax
from jax.experimental import pallas as pl
from jax.experimental.pallas import tpu as pltpu
from jax.experimental.pallas import tpu_sc as plsc


_CH = 128
_K = 2
_NBUF = 5
_LAG = 3


@functools.lru_cache(maxsize=None)
def _build(n_workers: int, n_cores: int, b_tot: int, vocab: int, d: int):
    b_per_w = b_tot // n_workers
    n_ch = b_per_w // _CH
    n_grp = n_ch // _K
    rounds = n_grp // _NBUF
    grp = _K * _CH
    mesh = plsc.VectorSubcoreMesh(core_axis_name="c", subcore_axis_name="s")

    scratch = [pltpu.VMEM((n_ch, _CH), jnp.int32)]
    scratch += [pltpu.VMEM((grp, d), jnp.float32) for _ in range(_NBUF)]
    scratch += [pltpu.SemaphoreType.DMA for _ in range(_NBUF)]

    @functools.partial(
        pl.kernel,
        mesh=mesh,
        out_type=jax.ShapeDtypeStruct((b_tot, d), jnp.float32),
        compiler_params=pltpu.CompilerParams(use_tc_tiling_on_sc=False),
        scratch_types=scratch,
    )
    def emb(x_hbm, table_hbm, out_hbm, idx_v, *rest):
        bufs = rest[:_NBUF]
        gsem = rest[_NBUF:]
        wid = lax.axis_index("s") * n_cores + lax.axis_index("c")
        base = wid * b_per_w
        pltpu.sync_copy(x_hbm.at[wid], idx_v)

        def g_copies(g, b):
            return [
                pltpu.make_async_copy(
                    table_hbm.at[idx_v.at[g * _K + j]],
                    bufs[b].at[pl.ds(j * _CH, _CH)], gsem[b])
                for j in range(_K)
            ]

        for b in range(_LAG):
            for cp in g_copies(b, b):
                cp.start()

        def body(r, carry):
            g0 = r * _NBUF
            for i in range(_NBUF):
                g = g0 + i
                fg = g + _LAG
                fb = (i + _LAG) % _NBUF

                @pl.when(fg < n_grp)
                def _():
                    for cp in g_copies(fg, fb):
                        cp.start()

                for cp in g_copies(g, i):
                    cp.wait()
                pltpu.sync_copy(bufs[i], out_hbm.at[pl.ds(base + g * grp, grp)])
            return carry

        lax.fori_loop(0, rounds, body, 0)

    return emb


def kernel(x, table):
    b, s = x.shape
    vocab, d = table.shape
    info = plsc.get_sparse_core_info()
    n_workers = info.num_cores * info.num_subcores
    b_tot = b * s
    b_per_w = b_tot // n_workers
    xf = x.reshape(n_workers, b_per_w // _CH, _CH)
    emb = _build(n_workers, info.num_cores, b_tot, vocab, d)
    out = emb(xf, table)
    return out.reshape(b, s, d)


# speedup vs baseline: 1.1159x; 1.0003x over previous
import functools

import jax
import jax.numpy as jnp
from jax import lax
from jax.experimental import pallas as pl
from jax.experimental.pallas import tpu as pltpu
from jax.experimental.pallas import tpu_sc as plsc


_CH = 128
_K = 2
_NBUF = 5
_LAG = 3


@functools.lru_cache(maxsize=None)
def _build(n_workers: int, n_cores: int, b_tot: int, vocab: int, d: int):
    b_per_w = b_tot // n_workers
    n_ch = b_per_w // _CH
    n_grp = n_ch // _K
    rounds = n_grp // _NBUF
    grp = _K * _CH
    mesh = plsc.VectorSubcoreMesh(core_axis_name="c", subcore_axis_name="s")

    scratch = [pltpu.VMEM((n_grp, grp), jnp.int32)]
    scratch += [pltpu.VMEM((grp, d), jnp.float32) for _ in range(_NBUF)]
    scratch += [pltpu.SemaphoreType.DMA for _ in range(_NBUF)]

    @functools.partial(
        pl.kernel,
        mesh=mesh,
        out_type=jax.ShapeDtypeStruct((b_tot, d), jnp.float32),
        compiler_params=pltpu.CompilerParams(use_tc_tiling_on_sc=False),
        scratch_types=scratch,
    )
    def emb(x_hbm, table_hbm, out_hbm, idx_v, *rest):
        bufs = rest[:_NBUF]
        gsem = rest[_NBUF:]
        wid = lax.axis_index("s") * n_cores + lax.axis_index("c")
        base = wid * b_per_w
        pltpu.sync_copy(x_hbm.at[wid], idx_v)

        def g_copies(g, b):
            return [
                pltpu.make_async_copy(
                    table_hbm.at[idx_v.at[g]],
                    bufs[b], gsem[b])
            ]

        for b in range(_LAG):
            for cp in g_copies(b, b):
                cp.start()

        def body(r, carry):
            g0 = r * _NBUF
            for i in range(_NBUF):
                g = g0 + i
                fg = g + _LAG
                fb = (i + _LAG) % _NBUF

                @pl.when(fg < n_grp)
                def _():
                    for cp in g_copies(fg, fb):
                        cp.start()

                for cp in g_copies(g, i):
                    cp.wait()
                pltpu.sync_copy(bufs[i], out_hbm.at[pl.ds(base + g * grp, grp)])
            return carry

        lax.fori_loop(0, rounds, body, 0)

    return emb


def kernel(x, table):
    b, s = x.shape
    vocab, d = table.shape
    info = plsc.get_sparse_core_info()
    n_workers = info.num_cores * info.num_subcores
    b_tot = b * s
    b_per_w = b_tot // n_workers
    xf = x.reshape(n_workers, b_per_w // (_K * _CH), _K * _CH)
    emb = _build(n_workers, info.num_cores, b_tot, vocab, d)
    out = emb(xf, table)
    return out.reshape(b, s, d)
